# baseline (speedup 1.0000x reference)
import functools

import jax
import jax.numpy as jnp
from jax import lax
from jax.experimental import pallas as pl
from jax.experimental.pallas import tpu as pltpu
from jax.experimental.pallas import tpu_sc as plsc

VOCAB = 1000000
D_MODEL = 64
B_ROWS = 4096
T_COLS = 200

_info = plsc.get_sparse_core_info()
NC = _info.num_cores
NS = _info.num_subcores
NW = NC * NS
BL = B_ROWS // NW
NB = 3
NG = T_COLS // NB
assert T_COLS % NB < NB


def _make_gather():
    mesh = plsc.VectorSubcoreMesh(core_axis_name="c", subcore_axis_name="s")

    scratch = {
        "idx_v": pltpu.VMEM((T_COLS, BL), jnp.int32),
        "bufs": [pltpu.VMEM((BL, D_MODEL), jnp.float32) for _ in range(NB)],
        "slabs": [pltpu.VMEM((8, 8, BL), jnp.float32) for _ in range(NB)],
        "gsem": pltpu.SemaphoreType.DMA((NB,)),
        "wsem": pltpu.SemaphoreType.DMA((NB,)),
    }

    @functools.partial(
        pl.kernel,
        mesh=mesh,
        out_type=jax.ShapeDtypeStruct((T_COLS, 8, NW, 8, BL), jnp.float32),
        scratch_types=scratch,
        compiler_params=pltpu.CompilerParams(
            use_tc_tiling_on_sc=False, needs_layout_passes=False),
    )
    def gather_kernel(idx_hbm, table_hbm, out_hbm, idx_v, bufs, slabs,
                      gsem, wsem):
        wid = lax.axis_index("s") * NC + lax.axis_index("c")
        pltpu.sync_copy(idx_hbm.at[:, pl.ds(wid * BL, BL)], idx_v)

        iota = lax.iota(jnp.int32, 16)

        def fire(slot, t):
            pltpu.async_copy(table_hbm.at[idx_v.at[t]], bufs[slot],
                             gsem.at[slot])

        def gather_wait(slot):
            pltpu.make_async_copy(table_hbm.at[idx_v.at[0]], bufs[slot],
                                  gsem.at[slot]).wait()

        def transpose(slot):
            buf, slab = bufs[slot], slabs[slot]
            for g in range(8):
                rows = g * 16 + iota
                for d in range(64):
                    cols = jnp.full((16,), d, jnp.int32)
                    slab[d // 8, d % 8, pl.ds(g * 16, 16)] = (
                        plsc.load_gather(buf, [rows, cols]))

        def wb_start(slot, t):
            pltpu.async_copy(slabs[slot], out_hbm.at[t].at[:, wid],
                             wsem.at[slot])

        def wb_wait(slot):
            pltpu.make_async_copy(slabs[slot], out_hbm.at[0].at[:, 0],
                                  wsem.at[slot]).wait()

        for b in range(NB - 1):
            fire(b, b)

        def group(g, carry):
            for b in range(NB):
                t = g * NB + b
                t_pre = t + NB - 1
                slot_pre = (b + NB - 1) % NB

                @pl.when(t_pre < T_COLS)
                def _():
                    fire(slot_pre, t_pre)

                gather_wait(b)

                @pl.when(t >= NB)
                def _():
                    wb_wait(b)

                transpose(b)
                wb_start(b, t)
            return carry

        lax.fori_loop(0, NG, group, 0, unroll=False)

        for r in range(T_COLS % NB):
            t = NG * NB + r
            gather_wait(r)
            wb_wait(r)
            transpose(r)
            wb_start(r, t)
        for b in range(NB):
            wb_wait(b)

    return gather_kernel


_gather = _make_gather()


def kernel(token_ids, table):
    idx_t = token_ids.T
    out5 = _gather(idx_t, table)
    out = jnp.transpose(out5, (2, 4, 0, 1, 3)).reshape(B_ROWS, T_COLS, D_MODEL)
    return out


# speedup vs baseline: 1.8312x; 1.8312x over previous
import functools

import jax
import jax.numpy as jnp
from jax import lax
from jax.experimental import pallas as pl
from jax.experimental.pallas import tpu as pltpu
from jax.experimental.pallas import tpu_sc as plsc

VOCAB = 1000000
D_MODEL = 64
B_ROWS = 4096
T_COLS = 200

_info = plsc.get_sparse_core_info()
NC = _info.num_cores
NS = _info.num_subcores
NW = NC * NS
BL = B_ROWS // NW
NB = 3
NG = T_COLS // NB
assert T_COLS % NB < NB


def _make_gather():
    mesh = plsc.VectorSubcoreMesh(core_axis_name="c", subcore_axis_name="s")

    scratch = {
        "idx_v": pltpu.VMEM((T_COLS, BL), jnp.int32),
        "bufs": [pltpu.VMEM((BL, D_MODEL), jnp.float32) for _ in range(NB)],
        "slabs": [pltpu.VMEM((8, 8, 133), jnp.float32) for _ in range(NB)],
        "gsem": pltpu.SemaphoreType.DMA((NB,)),
        "wsem": pltpu.SemaphoreType.DMA((NB,)),
    }

    @functools.partial(
        pl.kernel,
        mesh=mesh,
        out_type=jax.ShapeDtypeStruct((T_COLS, 8, NW, 8, BL), jnp.float32),
        scratch_types=scratch,
        compiler_params=pltpu.CompilerParams(
            use_tc_tiling_on_sc=False, needs_layout_passes=False),
    )
    def gather_kernel(idx_hbm, table_hbm, out_hbm, idx_v, bufs, slabs,
                      gsem, wsem):
        wid = lax.axis_index("s") * NC + lax.axis_index("c")
        pltpu.sync_copy(idx_hbm.at[:, pl.ds(wid * BL, BL)], idx_v)

        iota = lax.iota(jnp.int32, 16)

        def fire(slot, t):
            pltpu.async_copy(table_hbm.at[idx_v.at[t]], bufs[slot],
                             gsem.at[slot])

        def gather_wait(slot):
            pltpu.make_async_copy(table_hbm.at[idx_v.at[0]], bufs[slot],
                                  gsem.at[slot]).wait()

        def transpose(slot):
            buf, slab = bufs[slot], slabs[slot]
            for k in range(4):
                hi = (k * 16 + iota) // 8
                lo = (k * 16 + iota) % 8
                for bl in range(BL):
                    v = buf[bl, pl.ds(k * 16, 16)]
                    plsc.store_scatter(
                        slab, [hi, lo, jnp.full((16,), bl, jnp.int32)], v)

        def wb_start(slot, t):
            pltpu.async_copy(slabs[slot].at[:, :, pl.ds(0, BL)],
                             out_hbm.at[t].at[:, wid], wsem.at[slot])

        def wb_wait(slot):
            pltpu.make_async_copy(slabs[slot].at[:, :, pl.ds(0, BL)],
                                  out_hbm.at[0].at[:, 0],
                                  wsem.at[slot]).wait()

        for b in range(NB - 1):
            fire(b, b)

        def group(g, carry):
            for b in range(NB):
                t = g * NB + b
                t_pre = t + NB - 1
                slot_pre = (b + NB - 1) % NB

                @pl.when(t_pre < T_COLS)
                def _():
                    fire(slot_pre, t_pre)

                gather_wait(b)

                @pl.when(t >= NB)
                def _():
                    wb_wait(b)

                transpose(b)
                wb_start(b, t)
            return carry

        lax.fori_loop(0, NG, group, 0, unroll=False)

        for r in range(T_COLS % NB):
            t = NG * NB + r
            gather_wait(r)
            wb_wait(r)
            transpose(r)
            wb_start(r, t)
        for b in range(NB):
            wb_wait(b)

    return gather_kernel


_gather = _make_gather()


def kernel(token_ids, table):
    idx_t = token_ids.T
    out5 = _gather(idx_t, table)
    out = jnp.transpose(out5, (2, 4, 0, 1, 3)).reshape(B_ROWS, T_COLS, D_MODEL)
    return out


# speedup vs baseline: 2.7234x; 1.4872x over previous
import functools

import jax
import jax.numpy as jnp
from jax import lax
from jax.experimental import pallas as pl
from jax.experimental.pallas import tpu as pltpu
from jax.experimental.pallas import tpu_sc as plsc

VOCAB = 1000000
D_MODEL = 64
B_ROWS = 4096
T_COLS = 200

_info = plsc.get_sparse_core_info()
NC = _info.num_cores
NS = _info.num_subcores
NW = NC * NS
BL = B_ROWS // NW
NB = 3
NG = T_COLS // NB
assert T_COLS % NB < NB


def _make_gather():
    mesh = plsc.VectorSubcoreMesh(core_axis_name="c", subcore_axis_name="s")

    scratch = {
        "idx_v": pltpu.VMEM((T_COLS, BL), jnp.int32),
        "bufs": [pltpu.VMEM((BL, D_MODEL), jnp.float32) for _ in range(NB)],
        "slabs": [pltpu.VMEM((8, 8, 133), jnp.float32) for _ in range(NB)],
        "gsem": pltpu.SemaphoreType.DMA((NB,)),
        "wsem": pltpu.SemaphoreType.DMA((NB,)),
    }

    @functools.partial(
        pl.kernel,
        mesh=mesh,
        out_type=jax.ShapeDtypeStruct((T_COLS, 8, NW, 8, BL), jnp.float32),
        scratch_types=scratch,
        compiler_params=pltpu.CompilerParams(
            use_tc_tiling_on_sc=False, needs_layout_passes=False),
    )
    def gather_kernel(idx_hbm, table_hbm, out_hbm, idx_v, bufs, slabs,
                      gsem, wsem):
        wid = lax.axis_index("s") * NC + lax.axis_index("c")
        pltpu.sync_copy(idx_hbm.at[:, pl.ds(wid * BL, BL)], idx_v)

        iota = lax.iota(jnp.int32, 16)

        def fire(slot, t):
            pltpu.async_copy(table_hbm.at[idx_v.at[t]], bufs[slot],
                             gsem.at[slot])

        def gather_wait(slot):
            pltpu.make_async_copy(table_hbm.at[idx_v.at[0]], bufs[slot],
                                  gsem.at[slot]).wait()

        def transpose(slot):
            buf, slab = bufs[slot], slabs[slot]

            @plsc.parallel_loop(0, BL, step=1, unroll=8)
            def _(bl):
                blv = jnp.full((16,), bl, jnp.int32)
                for k in range(4):
                    hi = (k * 16 + iota) // 8
                    lo = (k * 16 + iota) % 8
                    v = buf[bl, pl.ds(k * 16, 16)]
                    plsc.store_scatter(slab, [hi, lo, blv], v)

        def wb_start(slot, t):
            pltpu.async_copy(slabs[slot].at[:, :, pl.ds(0, BL)],
                             out_hbm.at[t].at[:, wid], wsem.at[slot])

        def wb_wait(slot):
            pltpu.make_async_copy(slabs[slot].at[:, :, pl.ds(0, BL)],
                                  out_hbm.at[0].at[:, 0],
                                  wsem.at[slot]).wait()

        for b in range(NB - 1):
            fire(b, b)

        def group(g, carry):
            for b in range(NB):
                t = g * NB + b
                t_pre = t + NB - 1
                slot_pre = (b + NB - 1) % NB

                @pl.when(t_pre < T_COLS)
                def _():
                    fire(slot_pre, t_pre)

                gather_wait(b)

                @pl.when(t >= NB)
                def _():
                    wb_wait(b)

                transpose(b)
                wb_start(b, t)
            return carry

        lax.fori_loop(0, NG, group, 0, unroll=False)

        for r in range(T_COLS % NB):
            t = NG * NB + r
            gather_wait(r)
            wb_wait(r)
            transpose(r)
            wb_start(r, t)
        for b in range(NB):
            wb_wait(b)

    return gather_kernel


_gather = _make_gather()


def kernel(token_ids, table):
    idx_t = token_ids.T
    out5 = _gather(idx_t, table)
    out = jnp.transpose(out5, (2, 4, 0, 1, 3)).reshape(B_ROWS, T_COLS, D_MODEL)
    return out
